# Initial kernel scaffold; baseline (speedup 1.0000x reference)
import jax
import jax.numpy as jnp
from jax.experimental import pallas as pl


def kernel(x, edge_index, W1l, W1r, b1, W2l, W2r, b2):
    raise NotImplementedError("write your pallas kernel here")


# speedup vs baseline: 2.5308x; 2.5308x over previous
import jax
import jax.numpy as jnp
from jax import lax
from jax.experimental import pallas as pl
from jax.experimental.pallas import tpu as pltpu
from jax.experimental.pallas import tpu_sc as plsc

N = 10000
E = 320000
D = 128

NC = 2
NS = 16
L = 16

CHUNK = 128
RPI = 4
EC = E // CHUNK
SUPER = EC // RPI
KBASE = SUPER // NS
KREM = SUPER - KBASE * NS
HALF = N // NC
HPAD = HALF + 8
SSL = 312
SSL_LAST = 320

BN = 1000
BPH = HALF // BN


def _sc_agg_body(src2d, dst2d, y, zrows, out,
                 sidx0, sidx1, sidx2, sidx3,
                 didx0, didx1, didx2, didx3,
                 rows0, rows1, rows2, rows3,
                 acc, sem):
    sidxs = (sidx0, sidx1, sidx2, sidx3)
    didxs = (didx0, didx1, didx2, didx3)
    rowss = (rows0, rows1, rows2, rows3)
    c = lax.axis_index("c")
    s = lax.axis_index("s")
    base = c * HALF

    @pl.when(s < NS - 1)
    def _():
        pltpu.sync_copy(zrows.at[pl.ds(0, SSL)], acc.at[pl.ds(s * SSL, SSL)])

    @pl.when(s == NS - 1)
    def _():
        pltpu.sync_copy(zrows, acc.at[pl.ds((NS - 1) * SSL, SSL_LAST + 8)])

    plsc.subcore_barrier()

    nk = KBASE + jnp.where(s < KREM, 1, 0)

    def body(k, carry):
        rb = (k * NS + s) * RPI
        for j in range(RPI):
            pltpu.sync_copy(dst2d.at[rb + j, pl.ds(0, CHUNK)], didxs[j])
            pltpu.sync_copy(src2d.at[rb + j, pl.ds(0, CHUNK)], sidxs[j])
        descs = [
            pltpu.async_copy(y.at[sidxs[j]], rowss[j], sem)
            for j in range(RPI)
        ]

        trash = jnp.full((L,), HALF, jnp.int32)
        for j in range(RPI):
            for g in range(CHUNK // L):
                dv = didxs[j][pl.ds(g * L, L)] - base
                bad = (dv < 0) | (dv >= HALF)
                didxs[j][pl.ds(g * L, L)] = jnp.where(bad, trash, dv)

        for d in descs:
            d.wait()
        for j in range(RPI):
            pltpu.sync_copy(rowss[j], acc.at[didxs[j]], add=True)
        return carry

    lax.fori_loop(0, nk, body, 0)
    plsc.subcore_barrier()

    @pl.when(s < NS - 1)
    def _():
        pltpu.sync_copy(acc.at[pl.ds(s * SSL, SSL)],
                        out.at[c, pl.ds(s * SSL, SSL)])

    @pl.when(s == NS - 1)
    def _():
        pltpu.sync_copy(acc.at[pl.ds((NS - 1) * SSL, SSL_LAST)],
                        out.at[c, pl.ds((NS - 1) * SSL, SSL_LAST)])


_sc_agg = pl.kernel(
    _sc_agg_body,
    mesh=plsc.VectorSubcoreMesh(core_axis_name="c", subcore_axis_name="s"),
    out_type=jax.ShapeDtypeStruct((NC, HALF, D), jnp.float32),
    scratch_types=(
        [pltpu.VMEM((CHUNK,), jnp.int32) for _ in range(2 * RPI)]
        + [pltpu.VMEM((CHUNK, D), jnp.float32) for _ in range(RPI)]
        + [pltpu.VMEM_SHARED((HPAD, D), jnp.float32),
           pltpu.SemaphoreType.DMA]
    ),
)


def _lin1_body(x_ref, wl_ref, wr_ref, b_ref, y_ref, xr_ref):
    xb = x_ref[...]
    y_ref[...] = jnp.dot(xb, wl_ref[...], preferred_element_type=jnp.float32)
    xr_ref[...] = (
        jnp.dot(xb, wr_ref[...], preferred_element_type=jnp.float32)
        + b_ref[...]
    )


def _stage_body(p_ref, y1_ref, xr_ref, cnt_ref, wl_ref, wr_ref, b_ref,
                fc_ref, fr_ref, ytab2_ref, xr2_ref, cnt2_ref, t_ref):
    pb = p_ref[0]
    fc = fc_ref[...]
    fr = fr_ref[...]
    cnt2 = fc * (pb[:, :1] * jnp.ones((1, D), jnp.float32)) \
        + (1.0 - fc) * cnt_ref[...]
    cnt2_ref[...] = cnt2
    t = pb / jnp.maximum(cnt2[:, :1], 1.0) + xr_ref[...]
    t_ref[...] = t
    h = jnp.maximum(t, (1.0 - fr) * t)
    ynew = jnp.dot(h, wl_ref[...], preferred_element_type=jnp.float32)
    xrnew = (
        jnp.dot(h, wr_ref[...], preferred_element_type=jnp.float32)
        + b_ref[...]
    )
    ytab2_ref[...] = fc * y1_ref[...] + (1.0 - fc) * ynew
    xr2_ref[...] = fc * xr_ref[...] + (1.0 - fc) * xrnew


def _row_spec(shape):
    return pl.BlockSpec(shape, lambda i: (i, 0))


def _full_spec(shape):
    return pl.BlockSpec(shape, lambda i: (0, 0))


_tc_lin1 = pl.pallas_call(
    _lin1_body,
    grid=(N // BN,),
    in_specs=[_row_spec((BN, D)), _full_spec((D, D)), _full_spec((D, D)),
              _full_spec((1, D))],
    out_specs=[_row_spec((BN, D)), _row_spec((BN, D))],
    out_shape=[jax.ShapeDtypeStruct((N, D), jnp.float32),
               jax.ShapeDtypeStruct((N, D), jnp.float32)],
)

_tc_stage = pl.pallas_call(
    _stage_body,
    grid=(N // BN,),
    in_specs=[pl.BlockSpec((1, BN, D), lambda i: (i // BPH, i % BPH, 0)),
              _row_spec((BN, D)), _row_spec((BN, D)), _row_spec((BN, D)),
              _full_spec((D, D)), _full_spec((D, D)), _full_spec((1, D)),
              _full_spec((1, D)), _full_spec((1, D))],
    out_specs=[_row_spec((BN, D)), _row_spec((BN, D)), _row_spec((BN, D)),
               _row_spec((BN, D))],
    out_shape=[jax.ShapeDtypeStruct((N, D), jnp.float32),
               jax.ShapeDtypeStruct((N, D), jnp.float32),
               jax.ShapeDtypeStruct((N, D), jnp.float32),
               jax.ShapeDtypeStruct((N, D), jnp.float32)],
)


@jax.jit
def kernel(x, edge_index, W1l, W1r, b1, W2l, W2r, b2):
    src2d = edge_index[0].reshape(EC, CHUNK)
    dst2d = edge_index[1].reshape(EC, CHUNK)
    zrows = jnp.zeros((SSL_LAST + 8, D), jnp.float32)
    b1r = b1.reshape(1, D)
    b2r = b2.reshape(1, D)

    y1, xr1 = _tc_lin1(x, W1l, W1r, b1r)

    def step(i, carry):
        ytab, xr, cnt, _ = carry
        p = _sc_agg(src2d, dst2d, ytab, zrows)
        fc = jnp.where(i == 0, 1.0, 0.0) * jnp.ones((1, D), jnp.float32)
        fr = jnp.where(i == 1, 1.0, 0.0) * jnp.ones((1, D), jnp.float32)
        ytab2, xr2, cnt2, t = _tc_stage(p, y1, xr, cnt, W2l, W2r, b2r,
                                        fc, fr)
        return (ytab2, xr2, cnt2, t)

    dummy = jnp.zeros((N, D), jnp.float32)
    onestab = jnp.ones((N, D), jnp.float32)
    _, _, _, out = lax.fori_loop(0, 3, step, (onestab, xr1, dummy, dummy))
    return out


# speedup vs baseline: 3.4143x; 1.3491x over previous
import jax
import jax.numpy as jnp
from jax import lax
from jax.experimental import pallas as pl
from jax.experimental.pallas import tpu as pltpu
from jax.experimental.pallas import tpu_sc as plsc

N = 10000
E = 320000
D = 128

NC = 2
NS = 16
L = 16

CHUNK = 128
RPI = 2
EC = E // CHUNK
SUPER = EC // RPI
KBASE = SUPER // NS
KREM = SUPER - KBASE * NS
HALF = N // NC
HPAD = HALF + 8
SSL = 312
SSL_LAST = 320

BN = 1000
BPH = HALF // BN


def _sc_agg_body(src2d, dst2d, y, zrows, mode, out,
                 sidxa0, sidxa1, didxa0, didxa1, rowsa0, rowsa1,
                 sidxb0, sidxb1, didxb0, didxb1, rowsb0, rowsb1,
                 modev, acc, sema, semb):
    A = ((sidxa0, didxa0, rowsa0), (sidxa1, didxa1, rowsa1))
    B = ((sidxb0, didxb0, rowsb0), (sidxb1, didxb1, rowsb1))
    c = lax.axis_index("c")
    s = lax.axis_index("s")
    base = c * HALF

    @pl.when(s < NS - 1)
    def _():
        pltpu.sync_copy(zrows.at[pl.ds(0, SSL)], acc.at[pl.ds(s * SSL, SSL)])

    @pl.when(s == NS - 1)
    def _():
        pltpu.sync_copy(zrows, acc.at[pl.ds((NS - 1) * SSL, SSL_LAST + 8)])

    pltpu.sync_copy(mode, modev)
    m = modev[...][0]

    @pl.when(m == 0)
    def _():
        pltpu.sync_copy(y.at[pl.ds(0, CHUNK)], rowsa0)

    plsc.subcore_barrier()

    nk = KBASE + jnp.where(s < KREM, 1, 0)

    def load_issue(k, bufs, sem):
        rb = (k * NS + s) * RPI
        for j in range(RPI):
            pltpu.sync_copy(dst2d.at[rb + j, pl.ds(0, CHUNK)], bufs[j][1])

        @pl.when(m > 0)
        def _():
            for j in range(RPI):
                pltpu.sync_copy(src2d.at[rb + j, pl.ds(0, CHUNK)], bufs[j][0])
            for j in range(RPI):
                pltpu.async_copy(y.at[bufs[j][0]], bufs[j][2], sem)

    def remap(bufs):
        trash = jnp.full((L,), HALF, jnp.int32)
        for j in range(RPI):
            didx = bufs[j][1]
            for g in range(CHUNK // L):
                dv = didx[pl.ds(g * L, L)] - base
                bad = (dv < 0) | (dv >= HALF)
                didx[pl.ds(g * L, L)] = jnp.where(bad, trash, dv)

    def wait_scatter(bufs, sem):
        @pl.when(m > 0)
        def _():
            for j in range(RPI):
                pltpu.make_async_copy(y.at[bufs[j][0]], bufs[j][2], sem).wait()
            for j in range(RPI):
                pltpu.sync_copy(bufs[j][2], acc.at[bufs[j][1]], add=True)

        @pl.when(m == 0)
        def _():
            for j in range(RPI):
                pltpu.sync_copy(rowsa0, acc.at[bufs[j][1]], add=True)

    def body(k2, carry):
        ka = 2 * k2
        kb = 2 * k2 + 1
        load_issue(ka, A, sema)
        load_issue(kb, B, semb)
        remap(A)
        wait_scatter(A, sema)
        remap(B)
        wait_scatter(B, semb)
        return carry

    lax.fori_loop(0, nk // 2, body, 0)

    @pl.when(nk % 2 == 1)
    def _():
        load_issue(nk - 1, A, sema)
        remap(A)
        wait_scatter(A, sema)

    plsc.subcore_barrier()

    @pl.when(s < NS - 1)
    def _():
        pltpu.sync_copy(acc.at[pl.ds(s * SSL, SSL)],
                        out.at[c, pl.ds(s * SSL, SSL)])

    @pl.when(s == NS - 1)
    def _():
        pltpu.sync_copy(acc.at[pl.ds((NS - 1) * SSL, SSL_LAST)],
                        out.at[c, pl.ds((NS - 1) * SSL, SSL_LAST)])


_sc_agg = pl.kernel(
    _sc_agg_body,
    mesh=plsc.VectorSubcoreMesh(core_axis_name="c", subcore_axis_name="s"),
    out_type=jax.ShapeDtypeStruct((NC, HALF, D), jnp.float32),
    scratch_types=(
        ([pltpu.VMEM((CHUNK,), jnp.int32)] * 4
         + [pltpu.VMEM((CHUNK, D), jnp.float32)] * 2) * 2
        + [pltpu.VMEM((L,), jnp.int32),
           pltpu.VMEM_SHARED((HPAD, D), jnp.float32),
           pltpu.SemaphoreType.DMA, pltpu.SemaphoreType.DMA]
    ),
)


def _lin1_body(x_ref, wl_ref, wr_ref, b_ref, y_ref, xr_ref):
    xb = x_ref[...]
    y_ref[...] = jnp.dot(xb, wl_ref[...], preferred_element_type=jnp.float32)
    xr_ref[...] = (
        jnp.dot(xb, wr_ref[...], preferred_element_type=jnp.float32)
        + b_ref[...]
    )


def _stage_body(p_ref, y1_ref, xr_ref, cnt_ref, wl_ref, wr_ref, b_ref,
                fc_ref, fr_ref, ytab2_ref, xr2_ref, cnt2_ref, t_ref):
    pb = p_ref[0]
    fc = fc_ref[...]
    fr = fr_ref[...]
    cnt2 = fc * (pb[:, :1] * jnp.ones((1, D), jnp.float32)) \
        + (1.0 - fc) * cnt_ref[...]
    cnt2_ref[...] = cnt2
    t = pb / jnp.maximum(cnt2[:, :1], 1.0) + xr_ref[...]
    t_ref[...] = t
    h = jnp.maximum(t, (1.0 - fr) * t)
    ynew = jnp.dot(h, wl_ref[...], preferred_element_type=jnp.float32)
    xrnew = (
        jnp.dot(h, wr_ref[...], preferred_element_type=jnp.float32)
        + b_ref[...]
    )
    ytab2_ref[...] = fc * y1_ref[...] + (1.0 - fc) * ynew
    xr2_ref[...] = fc * xr_ref[...] + (1.0 - fc) * xrnew


def _row_spec(shape):
    return pl.BlockSpec(shape, lambda i: (i, 0))


def _full_spec(shape):
    return pl.BlockSpec(shape, lambda i: (0, 0))


_tc_lin1 = pl.pallas_call(
    _lin1_body,
    grid=(N // BN,),
    in_specs=[_row_spec((BN, D)), _full_spec((D, D)), _full_spec((D, D)),
              _full_spec((1, D))],
    out_specs=[_row_spec((BN, D)), _row_spec((BN, D))],
    out_shape=[jax.ShapeDtypeStruct((N, D), jnp.float32),
               jax.ShapeDtypeStruct((N, D), jnp.float32)],
)

_tc_stage = pl.pallas_call(
    _stage_body,
    grid=(N // BN,),
    in_specs=[pl.BlockSpec((1, BN, D), lambda i: (i // BPH, i % BPH, 0)),
              _row_spec((BN, D)), _row_spec((BN, D)), _row_spec((BN, D)),
              _full_spec((D, D)), _full_spec((D, D)), _full_spec((1, D)),
              _full_spec((1, D)), _full_spec((1, D))],
    out_specs=[_row_spec((BN, D)), _row_spec((BN, D)), _row_spec((BN, D)),
               _row_spec((BN, D))],
    out_shape=[jax.ShapeDtypeStruct((N, D), jnp.float32),
               jax.ShapeDtypeStruct((N, D), jnp.float32),
               jax.ShapeDtypeStruct((N, D), jnp.float32),
               jax.ShapeDtypeStruct((N, D), jnp.float32)],
)


@jax.jit
def kernel(x, edge_index, W1l, W1r, b1, W2l, W2r, b2):
    src2d = edge_index[0].reshape(EC, CHUNK)
    dst2d = edge_index[1].reshape(EC, CHUNK)
    zrows = jnp.zeros((SSL_LAST + 8, D), jnp.float32)
    b1r = b1.reshape(1, D)
    b2r = b2.reshape(1, D)

    y1, xr1 = _tc_lin1(x, W1l, W1r, b1r)

    def step(i, carry):
        ytab, xr, cnt, _ = carry
        mode = jnp.where(i == 0, 0, 1) * jnp.ones((L,), jnp.int32)
        p = _sc_agg(src2d, dst2d, ytab, zrows, mode)
        fc = jnp.where(i == 0, 1.0, 0.0) * jnp.ones((1, D), jnp.float32)
        fr = jnp.where(i == 1, 1.0, 0.0) * jnp.ones((1, D), jnp.float32)
        ytab2, xr2, cnt2, t = _tc_stage(p, y1, xr, cnt, W2l, W2r, b2r,
                                        fc, fr)
        return (ytab2, xr2, cnt2, t)

    dummy = jnp.zeros((N, D), jnp.float32)
    onestab = jnp.ones((N, D), jnp.float32)
    _, _, _, out = lax.fori_loop(0, 3, step, (onestab, xr1, dummy, dummy))
    return out


# speedup vs baseline: 3.7691x; 1.1039x over previous
import jax
import jax.numpy as jnp
from jax import lax
from jax.experimental import pallas as pl
from jax.experimental.pallas import tpu as pltpu
from jax.experimental.pallas import tpu_sc as plsc

N = 10000
E = 320000
D = 128

NC = 2
NS = 16
L = 16

CHUNK = 128
RPI = 2
EC = E // CHUNK
SUPER = EC // RPI
KBASE = SUPER // NS
KREM = SUPER - KBASE * NS
HALF = N // NC
HPAD = HALF + 8
SSL = 312
SSL_LAST = 320

BN = 1000
BPH = HALF // BN


def _sc_agg_body(src2d, dst2d, y, zrows, mode, out,
                 sidxa0, sidxa1, didxa0, didxa1, rowsa0, rowsa1,
                 sidxb0, sidxb1, didxb0, didxb1, rowsb0, rowsb1,
                 modev, acc, sema, semb):
    A = ((sidxa0, didxa0, rowsa0), (sidxa1, didxa1, rowsa1))
    B = ((sidxb0, didxb0, rowsb0), (sidxb1, didxb1, rowsb1))
    c = lax.axis_index("c")
    s = lax.axis_index("s")
    base = c * HALF

    @pl.when(s < NS - 1)
    def _():
        pltpu.sync_copy(zrows.at[pl.ds(0, SSL)], acc.at[pl.ds(s * SSL, SSL)])

    @pl.when(s == NS - 1)
    def _():
        pltpu.sync_copy(zrows, acc.at[pl.ds((NS - 1) * SSL, SSL_LAST + 8)])

    pltpu.sync_copy(mode, modev)
    m = modev[...][0]

    @pl.when(m == 0)
    def _():
        pltpu.sync_copy(y.at[pl.ds(0, CHUNK)], rowsa0)

    plsc.subcore_barrier()

    nk = KBASE + jnp.where(s < KREM, 1, 0)

    def load_issue(k, bufs, sem):
        rb = (k * NS + s) * RPI
        for j in range(RPI):
            pltpu.sync_copy(dst2d.at[rb + j, pl.ds(0, CHUNK)], bufs[j][1])

        @pl.when(m > 0)
        def _():
            for j in range(RPI):
                pltpu.sync_copy(src2d.at[rb + j, pl.ds(0, CHUNK)], bufs[j][0])
            for j in range(RPI):
                pltpu.async_copy(y.at[bufs[j][0]], bufs[j][2], sem)

    def remap(bufs):
        for j in range(RPI):
            didx = bufs[j][1]
            for g in range(CHUNK // L):
                trash = jnp.full((L,), HALF + (g % 8), jnp.int32)
                dv = didx[pl.ds(g * L, L)] - base
                bad = (dv < 0) | (dv >= HALF)
                didx[pl.ds(g * L, L)] = jnp.where(bad, trash, dv)

    def wait_scatter(bufs, sem):
        @pl.when(m > 0)
        def _():
            for j in range(RPI):
                pltpu.make_async_copy(y.at[bufs[j][0]], bufs[j][2], sem).wait()
            for j in range(RPI):
                pltpu.sync_copy(bufs[j][2], acc.at[bufs[j][1]], add=True)

        @pl.when(m == 0)
        def _():
            for j in range(RPI):
                pltpu.sync_copy(rowsa0, acc.at[bufs[j][1]], add=True)

    def body(k2, carry):
        ka = 2 * k2
        kb = 2 * k2 + 1
        load_issue(ka, A, sema)
        load_issue(kb, B, semb)
        remap(A)
        wait_scatter(A, sema)
        remap(B)
        wait_scatter(B, semb)
        return carry

    lax.fori_loop(0, nk // 2, body, 0)

    @pl.when(nk % 2 == 1)
    def _():
        load_issue(nk - 1, A, sema)
        remap(A)
        wait_scatter(A, sema)

    plsc.subcore_barrier()

    @pl.when(s < NS - 1)
    def _():
        pltpu.sync_copy(acc.at[pl.ds(s * SSL, SSL)],
                        out.at[c, pl.ds(s * SSL, SSL)])

    @pl.when(s == NS - 1)
    def _():
        pltpu.sync_copy(acc.at[pl.ds((NS - 1) * SSL, SSL_LAST)],
                        out.at[c, pl.ds((NS - 1) * SSL, SSL_LAST)])


_sc_agg = pl.kernel(
    _sc_agg_body,
    mesh=plsc.VectorSubcoreMesh(core_axis_name="c", subcore_axis_name="s"),
    out_type=jax.ShapeDtypeStruct((NC, HALF, D), jnp.float32),
    scratch_types=(
        ([pltpu.VMEM((CHUNK,), jnp.int32)] * 4
         + [pltpu.VMEM((CHUNK, D), jnp.float32)] * 2) * 2
        + [pltpu.VMEM((L,), jnp.int32),
           pltpu.VMEM_SHARED((HPAD, D), jnp.float32),
           pltpu.SemaphoreType.DMA, pltpu.SemaphoreType.DMA]
    ),
)


def _lin1_body(x_ref, wl_ref, wr_ref, b_ref, y_ref, xr_ref):
    xb = x_ref[...]
    y_ref[...] = jnp.dot(xb, wl_ref[...], preferred_element_type=jnp.float32)
    xr_ref[...] = (
        jnp.dot(xb, wr_ref[...], preferred_element_type=jnp.float32)
        + b_ref[...]
    )


def _stage_body(p_ref, y1_ref, xr_ref, cnt_ref, wl_ref, wr_ref, b_ref,
                fc_ref, fr_ref, ytab2_ref, xr2_ref, cnt2_ref, t_ref):
    pb = p_ref[0]
    fc = fc_ref[...]
    fr = fr_ref[...]
    cnt2 = fc * (pb[:, :1] * jnp.ones((1, D), jnp.float32)) \
        + (1.0 - fc) * cnt_ref[...]
    cnt2_ref[...] = cnt2
    t = pb / jnp.maximum(cnt2[:, :1], 1.0) + xr_ref[...]
    t_ref[...] = t
    h = jnp.maximum(t, (1.0 - fr) * t)
    ynew = jnp.dot(h, wl_ref[...], preferred_element_type=jnp.float32)
    xrnew = (
        jnp.dot(h, wr_ref[...], preferred_element_type=jnp.float32)
        + b_ref[...]
    )
    ytab2_ref[...] = fc * y1_ref[...] + (1.0 - fc) * ynew
    xr2_ref[...] = fc * xr_ref[...] + (1.0 - fc) * xrnew


def _row_spec(shape):
    return pl.BlockSpec(shape, lambda i: (i, 0))


def _full_spec(shape):
    return pl.BlockSpec(shape, lambda i: (0, 0))


_tc_lin1 = pl.pallas_call(
    _lin1_body,
    grid=(N // BN,),
    in_specs=[_row_spec((BN, D)), _full_spec((D, D)), _full_spec((D, D)),
              _full_spec((1, D))],
    out_specs=[_row_spec((BN, D)), _row_spec((BN, D))],
    out_shape=[jax.ShapeDtypeStruct((N, D), jnp.float32),
               jax.ShapeDtypeStruct((N, D), jnp.float32)],
)

_tc_stage = pl.pallas_call(
    _stage_body,
    grid=(N // BN,),
    in_specs=[pl.BlockSpec((1, BN, D), lambda i: (i // BPH, i % BPH, 0)),
              _row_spec((BN, D)), _row_spec((BN, D)), _row_spec((BN, D)),
              _full_spec((D, D)), _full_spec((D, D)), _full_spec((1, D)),
              _full_spec((1, D)), _full_spec((1, D))],
    out_specs=[_row_spec((BN, D)), _row_spec((BN, D)), _row_spec((BN, D)),
               _row_spec((BN, D))],
    out_shape=[jax.ShapeDtypeStruct((N, D), jnp.float32),
               jax.ShapeDtypeStruct((N, D), jnp.float32),
               jax.ShapeDtypeStruct((N, D), jnp.float32),
               jax.ShapeDtypeStruct((N, D), jnp.float32)],
)


@jax.jit
def kernel(x, edge_index, W1l, W1r, b1, W2l, W2r, b2):
    src2d = edge_index[0].reshape(EC, CHUNK)
    dst2d = edge_index[1].reshape(EC, CHUNK)
    zrows = jnp.zeros((SSL_LAST + 8, D), jnp.float32)
    b1r = b1.reshape(1, D)
    b2r = b2.reshape(1, D)

    y1, xr1 = _tc_lin1(x, W1l, W1r, b1r)

    def step(i, carry):
        ytab, xr, cnt, _ = carry
        mode = jnp.where(i == 0, 0, 1) * jnp.ones((L,), jnp.int32)
        p = _sc_agg(src2d, dst2d, ytab, zrows, mode)
        fc = jnp.where(i == 0, 1.0, 0.0) * jnp.ones((1, D), jnp.float32)
        fr = jnp.where(i == 1, 1.0, 0.0) * jnp.ones((1, D), jnp.float32)
        ytab2, xr2, cnt2, t = _tc_stage(p, y1, xr, cnt, W2l, W2r, b2r,
                                        fc, fr)
        return (ytab2, xr2, cnt2, t)

    dummy = jnp.zeros((N, D), jnp.float32)
    onestab = jnp.ones((N, D), jnp.float32)
    _, _, _, out = lax.fori_loop(0, 3, step, (onestab, xr1, dummy, dummy))
    return out


# speedup vs baseline: 4.1094x; 1.0903x over previous
import jax
import jax.numpy as jnp
from jax import lax
from jax.experimental import pallas as pl
from jax.experimental.pallas import tpu as pltpu
from jax.experimental.pallas import tpu_sc as plsc

N = 10000
E = 320000
D = 128

NC = 2
NS = 16
L = 16

CHUNK = 128
RPI = 2
EC = E // CHUNK
SUPER = EC // RPI
KBASE = SUPER // NS
KREM = SUPER - KBASE * NS
HALF = N // NC
HPAD = HALF + 8
SSL = 312
SSL_LAST = 320

BN = 1000
BPH = HALF // BN


def _sc_agg_body(src2d, dst2d, y, zrows, mode, out,
                 sidxa0, sidxa1, didxa0, didxa1, rowsa0, rowsa1,
                 sidxb0, sidxb1, didxb0, didxb1, rowsb0, rowsb1,
                 modev, acc, sema, semb, scsema, scsemb):
    A = ((sidxa0, didxa0, rowsa0), (sidxa1, didxa1, rowsa1))
    B = ((sidxb0, didxb0, rowsb0), (sidxb1, didxb1, rowsb1))
    c = lax.axis_index("c")
    s = lax.axis_index("s")
    base = c * HALF

    @pl.when(s < NS - 1)
    def _():
        pltpu.sync_copy(zrows.at[pl.ds(0, SSL)], acc.at[pl.ds(s * SSL, SSL)])

    @pl.when(s == NS - 1)
    def _():
        pltpu.sync_copy(zrows, acc.at[pl.ds((NS - 1) * SSL, SSL_LAST + 8)])

    pltpu.sync_copy(mode, modev)
    m = modev[...][0]

    @pl.when(m == 0)
    def _():
        pltpu.sync_copy(y.at[pl.ds(0, CHUNK)], rowsa0)

    plsc.subcore_barrier()

    nk = KBASE + jnp.where(s < KREM, 1, 0)

    def load_issue(k, bufs, sem):
        rb = (k * NS + s) * RPI
        for j in range(RPI):
            pltpu.sync_copy(dst2d.at[rb + j, pl.ds(0, CHUNK)], bufs[j][1])

        @pl.when(m > 0)
        def _():
            for j in range(RPI):
                pltpu.sync_copy(src2d.at[rb + j, pl.ds(0, CHUNK)], bufs[j][0])
            for j in range(RPI):
                pltpu.async_copy(y.at[bufs[j][0]], bufs[j][2], sem)

    def remap(bufs):
        for j in range(RPI):
            didx = bufs[j][1]
            for g in range(CHUNK // L):
                trash = jnp.full((L,), HALF + (g % 8), jnp.int32)
                dv = didx[pl.ds(g * L, L)] - base
                bad = (dv < 0) | (dv >= HALF)
                didx[pl.ds(g * L, L)] = jnp.where(bad, trash, dv)

    def wait_scatter(bufs, sem, scsem):
        @pl.when(m > 0)
        def _():
            for j in range(RPI):
                pltpu.make_async_copy(y.at[bufs[j][0]], bufs[j][2], sem).wait()
            for j in range(RPI):
                pltpu.async_copy(bufs[j][2], acc.at[bufs[j][1]], scsem,
                                 add=True)

        @pl.when(m == 0)
        def _():
            for j in range(RPI):
                pltpu.sync_copy(rowsa0, acc.at[bufs[j][1]], add=True)

    def drain_scatter(bufs, scsem):
        @pl.when(m > 0)
        def _():
            for j in range(RPI):
                pltpu.make_async_copy(bufs[j][2], acc.at[bufs[j][1]],
                                      scsem).wait()

    def body(k2, carry):
        @pl.when(k2 > 0)
        def _():
            drain_scatter(A, scsema)
        load_issue(2 * k2, A, sema)

        @pl.when(k2 > 0)
        def _():
            drain_scatter(B, scsemb)
        load_issue(2 * k2 + 1, B, semb)
        remap(A)
        wait_scatter(A, sema, scsema)
        remap(B)
        wait_scatter(B, semb, scsemb)
        return carry

    nk2 = nk // 2
    lax.fori_loop(0, nk2, body, 0)

    @pl.when(nk2 > 0)
    def _():
        drain_scatter(A, scsema)
        drain_scatter(B, scsemb)

    @pl.when(nk % 2 == 1)
    def _():
        load_issue(nk - 1, A, sema)
        remap(A)

        @pl.when(m > 0)
        def _():
            for j in range(RPI):
                pltpu.make_async_copy(y.at[A[j][0]], A[j][2], sema).wait()
            for j in range(RPI):
                pltpu.sync_copy(A[j][2], acc.at[A[j][1]], add=True)

        @pl.when(m == 0)
        def _():
            for j in range(RPI):
                pltpu.sync_copy(rowsa0, acc.at[A[j][1]], add=True)

    plsc.subcore_barrier()

    @pl.when(s < NS - 1)
    def _():
        pltpu.sync_copy(acc.at[pl.ds(s * SSL, SSL)],
                        out.at[c, pl.ds(s * SSL, SSL)])

    @pl.when(s == NS - 1)
    def _():
        pltpu.sync_copy(acc.at[pl.ds((NS - 1) * SSL, SSL_LAST)],
                        out.at[c, pl.ds((NS - 1) * SSL, SSL_LAST)])


_sc_agg = pl.kernel(
    _sc_agg_body,
    mesh=plsc.VectorSubcoreMesh(core_axis_name="c", subcore_axis_name="s"),
    out_type=jax.ShapeDtypeStruct((NC, HALF, D), jnp.float32),
    scratch_types=(
        ([pltpu.VMEM((CHUNK,), jnp.int32)] * 4
         + [pltpu.VMEM((CHUNK, D), jnp.float32)] * 2) * 2
        + [pltpu.VMEM((L,), jnp.int32),
           pltpu.VMEM_SHARED((HPAD, D), jnp.float32),
           pltpu.SemaphoreType.DMA, pltpu.SemaphoreType.DMA,
           pltpu.SemaphoreType.DMA, pltpu.SemaphoreType.DMA]
    ),
)


def _lin1_body(x_ref, wl_ref, wr_ref, b_ref, y_ref, xr_ref):
    xb = x_ref[...]
    y_ref[...] = jnp.dot(xb, wl_ref[...], preferred_element_type=jnp.float32)
    xr_ref[...] = (
        jnp.dot(xb, wr_ref[...], preferred_element_type=jnp.float32)
        + b_ref[...]
    )


def _stage_body(p_ref, y1_ref, xr_ref, cnt_ref, wl_ref, wr_ref, b_ref,
                fc_ref, fr_ref, ytab2_ref, xr2_ref, cnt2_ref, t_ref):
    pb = p_ref[0]
    fc = fc_ref[...]
    fr = fr_ref[...]
    cnt2 = fc * (pb[:, :1] * jnp.ones((1, D), jnp.float32)) \
        + (1.0 - fc) * cnt_ref[...]
    cnt2_ref[...] = cnt2
    t = pb / jnp.maximum(cnt2[:, :1], 1.0) + xr_ref[...]
    t_ref[...] = t
    h = jnp.maximum(t, (1.0 - fr) * t)
    ynew = jnp.dot(h, wl_ref[...], preferred_element_type=jnp.float32)
    xrnew = (
        jnp.dot(h, wr_ref[...], preferred_element_type=jnp.float32)
        + b_ref[...]
    )
    ytab2_ref[...] = fc * y1_ref[...] + (1.0 - fc) * ynew
    xr2_ref[...] = fc * xr_ref[...] + (1.0 - fc) * xrnew


def _row_spec(shape):
    return pl.BlockSpec(shape, lambda i: (i, 0))


def _full_spec(shape):
    return pl.BlockSpec(shape, lambda i: (0, 0))


_tc_lin1 = pl.pallas_call(
    _lin1_body,
    grid=(N // BN,),
    in_specs=[_row_spec((BN, D)), _full_spec((D, D)), _full_spec((D, D)),
              _full_spec((1, D))],
    out_specs=[_row_spec((BN, D)), _row_spec((BN, D))],
    out_shape=[jax.ShapeDtypeStruct((N, D), jnp.float32),
               jax.ShapeDtypeStruct((N, D), jnp.float32)],
)

_tc_stage = pl.pallas_call(
    _stage_body,
    grid=(N // BN,),
    in_specs=[pl.BlockSpec((1, BN, D), lambda i: (i // BPH, i % BPH, 0)),
              _row_spec((BN, D)), _row_spec((BN, D)), _row_spec((BN, D)),
              _full_spec((D, D)), _full_spec((D, D)), _full_spec((1, D)),
              _full_spec((1, D)), _full_spec((1, D))],
    out_specs=[_row_spec((BN, D)), _row_spec((BN, D)), _row_spec((BN, D)),
               _row_spec((BN, D))],
    out_shape=[jax.ShapeDtypeStruct((N, D), jnp.float32),
               jax.ShapeDtypeStruct((N, D), jnp.float32),
               jax.ShapeDtypeStruct((N, D), jnp.float32),
               jax.ShapeDtypeStruct((N, D), jnp.float32)],
)


@jax.jit
def kernel(x, edge_index, W1l, W1r, b1, W2l, W2r, b2):
    src2d = edge_index[0].reshape(EC, CHUNK)
    dst2d = edge_index[1].reshape(EC, CHUNK)
    zrows = jnp.zeros((SSL_LAST + 8, D), jnp.float32)
    b1r = b1.reshape(1, D)
    b2r = b2.reshape(1, D)

    y1, xr1 = _tc_lin1(x, W1l, W1r, b1r)

    def step(i, carry):
        ytab, xr, cnt, _ = carry
        mode = jnp.where(i == 0, 0, 1) * jnp.ones((L,), jnp.int32)
        p = _sc_agg(src2d, dst2d, ytab, zrows, mode)
        fc = jnp.where(i == 0, 1.0, 0.0) * jnp.ones((1, D), jnp.float32)
        fr = jnp.where(i == 1, 1.0, 0.0) * jnp.ones((1, D), jnp.float32)
        ytab2, xr2, cnt2, t = _tc_stage(p, y1, xr, cnt, W2l, W2r, b2r,
                                        fc, fr)
        return (ytab2, xr2, cnt2, t)

    dummy = jnp.zeros((N, D), jnp.float32)
    onestab = jnp.ones((N, D), jnp.float32)
    _, _, _, out = lax.fori_loop(0, 3, step, (onestab, xr1, dummy, dummy))
    return out


# speedup vs baseline: 4.1503x; 1.0099x over previous
import jax
import jax.numpy as jnp
from jax import lax
from jax.experimental import pallas as pl
from jax.experimental.pallas import tpu as pltpu
from jax.experimental.pallas import tpu_sc as plsc

N = 10000
E = 320000
D = 128

NC = 2
NS = 16
L = 16

CHUNK = 128
RPI = 2
EC = E // CHUNK
SUPER = EC // RPI
KBASE = SUPER // NS
KREM = SUPER - KBASE * NS
HALF = N // NC
HPAD = HALF + 8
SSL = 312
SSL_LAST = 320

BN = 1000
BPH = HALF // BN


def _sc_agg_body(src2d, dst2d, y, zrows, mode, out,
                 sidxa0, sidxa1, didxa0, didxa1, rowsa0, rowsa1,
                 sidxb0, sidxb1, didxb0, didxb1, rowsb0, rowsb1,
                 modev, acc, sema, semb, scsema, scsemb):
    A = ((sidxa0, didxa0, rowsa0), (sidxa1, didxa1, rowsa1))
    B = ((sidxb0, didxb0, rowsb0), (sidxb1, didxb1, rowsb1))
    c = lax.axis_index("c")
    s = lax.axis_index("s")
    base = c * HALF

    @pl.when(s < NS - 1)
    def _():
        pltpu.sync_copy(zrows.at[pl.ds(0, SSL)], acc.at[pl.ds(s * SSL, SSL)])

    @pl.when(s == NS - 1)
    def _():
        pltpu.sync_copy(zrows, acc.at[pl.ds((NS - 1) * SSL, SSL_LAST + 8)])

    pltpu.sync_copy(mode, modev)
    m = modev[...][0]

    @pl.when(m == 0)
    def _():
        pltpu.sync_copy(y.at[pl.ds(0, CHUNK)], rowsa0)

    plsc.subcore_barrier()

    nk = KBASE + jnp.where(s < KREM, 1, 0)

    def load_issue(k, bufs, sem):
        rb = (k * NS + s) * RPI
        for j in range(RPI):
            pltpu.sync_copy(dst2d.at[rb + j, pl.ds(0, CHUNK)], bufs[j][1])

        @pl.when(m > 0)
        def _():
            for j in range(RPI):
                pltpu.sync_copy(src2d.at[rb + j, pl.ds(0, CHUNK)], bufs[j][0])
            for j in range(RPI):
                pltpu.async_copy(y.at[bufs[j][0]], bufs[j][2], sem)

    def remap(bufs):
        for j in range(RPI):
            didx = bufs[j][1]
            for g in range(CHUNK // L):
                trash = jnp.full((L,), HALF + (g % 8), jnp.int32)
                dv = didx[pl.ds(g * L, L)] - base
                bad = (dv < 0) | (dv >= HALF)
                didx[pl.ds(g * L, L)] = jnp.where(bad, trash, dv)

    def wait_scatter(bufs, sem, scsem):
        @pl.when(m > 0)
        def _():
            for j in range(RPI):
                pltpu.make_async_copy(y.at[bufs[j][0]], bufs[j][2], sem).wait()
            for j in range(RPI):
                pltpu.async_copy(bufs[j][2], acc.at[bufs[j][1]], scsem,
                                 add=True)

        @pl.when(m == 0)
        def _():
            for j in range(RPI):
                pltpu.async_copy(rowsa0, acc.at[bufs[j][1]], scsem, add=True)

    def drain_scatter(bufs, scsem):
        @pl.when(m > 0)
        def _():
            for j in range(RPI):
                pltpu.make_async_copy(bufs[j][2], acc.at[bufs[j][1]],
                                      scsem).wait()

        @pl.when(m == 0)
        def _():
            for j in range(RPI):
                pltpu.make_async_copy(rowsa0, acc.at[bufs[j][1]],
                                      scsem).wait()

    def body(k2, carry):
        @pl.when(k2 > 0)
        def _():
            drain_scatter(A, scsema)
        load_issue(2 * k2, A, sema)

        @pl.when(k2 > 0)
        def _():
            drain_scatter(B, scsemb)
        load_issue(2 * k2 + 1, B, semb)
        remap(A)
        wait_scatter(A, sema, scsema)
        remap(B)
        wait_scatter(B, semb, scsemb)
        return carry

    nk2 = nk // 2
    lax.fori_loop(0, nk2, body, 0)

    @pl.when(nk2 > 0)
    def _():
        drain_scatter(A, scsema)
        drain_scatter(B, scsemb)

    @pl.when(nk % 2 == 1)
    def _():
        load_issue(nk - 1, A, sema)
        remap(A)

        @pl.when(m > 0)
        def _():
            for j in range(RPI):
                pltpu.make_async_copy(y.at[A[j][0]], A[j][2], sema).wait()
            for j in range(RPI):
                pltpu.sync_copy(A[j][2], acc.at[A[j][1]], add=True)

        @pl.when(m == 0)
        def _():
            for j in range(RPI):
                pltpu.sync_copy(rowsa0, acc.at[A[j][1]], add=True)

    plsc.subcore_barrier()

    @pl.when(s < NS - 1)
    def _():
        pltpu.sync_copy(acc.at[pl.ds(s * SSL, SSL)],
                        out.at[c, pl.ds(s * SSL, SSL)])

    @pl.when(s == NS - 1)
    def _():
        pltpu.sync_copy(acc.at[pl.ds((NS - 1) * SSL, SSL_LAST)],
                        out.at[c, pl.ds((NS - 1) * SSL, SSL_LAST)])


_sc_agg = pl.kernel(
    _sc_agg_body,
    mesh=plsc.VectorSubcoreMesh(core_axis_name="c", subcore_axis_name="s"),
    out_type=jax.ShapeDtypeStruct((NC, HALF, D), jnp.float32),
    scratch_types=(
        ([pltpu.VMEM((CHUNK,), jnp.int32)] * 4
         + [pltpu.VMEM((CHUNK, D), jnp.float32)] * 2) * 2
        + [pltpu.VMEM((L,), jnp.int32),
           pltpu.VMEM_SHARED((HPAD, D), jnp.float32),
           pltpu.SemaphoreType.DMA, pltpu.SemaphoreType.DMA,
           pltpu.SemaphoreType.DMA, pltpu.SemaphoreType.DMA]
    ),
)


def _lin1_body(x_ref, wl_ref, wr_ref, b_ref, y_ref, xr_ref):
    xb = x_ref[...]
    y_ref[...] = jnp.dot(xb, wl_ref[...], preferred_element_type=jnp.float32)
    xr_ref[...] = (
        jnp.dot(xb, wr_ref[...], preferred_element_type=jnp.float32)
        + b_ref[...]
    )


def _stage_body(p_ref, y1_ref, xr_ref, cnt_ref, wl_ref, wr_ref, b_ref,
                fc_ref, fr_ref, ytab2_ref, xr2_ref, cnt2_ref, t_ref):
    pb = p_ref[0]
    fc = fc_ref[...]
    fr = fr_ref[...]
    cnt2 = fc * (pb[:, :1] * jnp.ones((1, D), jnp.float32)) \
        + (1.0 - fc) * cnt_ref[...]
    cnt2_ref[...] = cnt2
    t = pb / jnp.maximum(cnt2[:, :1], 1.0) + xr_ref[...]
    t_ref[...] = t
    h = jnp.maximum(t, (1.0 - fr) * t)
    ynew = jnp.dot(h, wl_ref[...], preferred_element_type=jnp.float32)
    xrnew = (
        jnp.dot(h, wr_ref[...], preferred_element_type=jnp.float32)
        + b_ref[...]
    )
    ytab2_ref[...] = fc * y1_ref[...] + (1.0 - fc) * ynew
    xr2_ref[...] = fc * xr_ref[...] + (1.0 - fc) * xrnew


def _row_spec(shape):
    return pl.BlockSpec(shape, lambda i: (i, 0))


def _full_spec(shape):
    return pl.BlockSpec(shape, lambda i: (0, 0))


_tc_lin1 = pl.pallas_call(
    _lin1_body,
    grid=(N // BN,),
    in_specs=[_row_spec((BN, D)), _full_spec((D, D)), _full_spec((D, D)),
              _full_spec((1, D))],
    out_specs=[_row_spec((BN, D)), _row_spec((BN, D))],
    out_shape=[jax.ShapeDtypeStruct((N, D), jnp.float32),
               jax.ShapeDtypeStruct((N, D), jnp.float32)],
)

_tc_stage = pl.pallas_call(
    _stage_body,
    grid=(N // BN,),
    in_specs=[pl.BlockSpec((1, BN, D), lambda i: (i // BPH, i % BPH, 0)),
              _row_spec((BN, D)), _row_spec((BN, D)), _row_spec((BN, D)),
              _full_spec((D, D)), _full_spec((D, D)), _full_spec((1, D)),
              _full_spec((1, D)), _full_spec((1, D))],
    out_specs=[_row_spec((BN, D)), _row_spec((BN, D)), _row_spec((BN, D)),
               _row_spec((BN, D))],
    out_shape=[jax.ShapeDtypeStruct((N, D), jnp.float32),
               jax.ShapeDtypeStruct((N, D), jnp.float32),
               jax.ShapeDtypeStruct((N, D), jnp.float32),
               jax.ShapeDtypeStruct((N, D), jnp.float32)],
)


@jax.jit
def kernel(x, edge_index, W1l, W1r, b1, W2l, W2r, b2):
    src2d = edge_index[0].reshape(EC, CHUNK)
    dst2d = edge_index[1].reshape(EC, CHUNK)
    zrows = jnp.zeros((SSL_LAST + 8, D), jnp.float32)
    b1r = b1.reshape(1, D)
    b2r = b2.reshape(1, D)

    y1, xr1 = _tc_lin1(x, W1l, W1r, b1r)

    def step(i, carry):
        ytab, xr, cnt, _ = carry
        mode = jnp.where(i == 0, 0, 1) * jnp.ones((L,), jnp.int32)
        p = _sc_agg(src2d, dst2d, ytab, zrows, mode)
        fc = jnp.where(i == 0, 1.0, 0.0) * jnp.ones((1, D), jnp.float32)
        fr = jnp.where(i == 1, 1.0, 0.0) * jnp.ones((1, D), jnp.float32)
        ytab2, xr2, cnt2, t = _tc_stage(p, y1, xr, cnt, W2l, W2r, b2r,
                                        fc, fr)
        return (ytab2, xr2, cnt2, t)

    dummy = jnp.zeros((N, D), jnp.float32)
    onestab = jnp.ones((N, D), jnp.float32)
    _, _, _, out = lax.fori_loop(0, 3, step, (onestab, xr1, dummy, dummy))
    return out


# speedup vs baseline: 4.1861x; 1.0086x over previous
import jax
import jax.numpy as jnp
from jax import lax
from jax.experimental import pallas as pl
from jax.experimental.pallas import tpu as pltpu
from jax.experimental.pallas import tpu_sc as plsc

N = 10000
E = 320000
D = 128

NC = 2
NS = 16
L = 16

CHUNK = 128
RPI = 2
EC = E // CHUNK
SUPER = EC // RPI
KBASE = SUPER // NS
KREM = SUPER - KBASE * NS
HALF = N // NC
HPAD = HALF + 8
SSL = 312
SSL_LAST = 320

BN = 1000
BPH = HALF // BN


def _sc_agg_body(src2d, dst2d, y, zrows, mode, out,
                 sidxa0, sidxa1, didxa0, didxa1, rowsa0, rowsa1,
                 sidxb0, sidxb1, didxb0, didxb1, rowsb0, rowsb1,
                 modev, acc, sema, semb, scsema, scsemb):
    A = ((sidxa0, didxa0, rowsa0), (sidxa1, didxa1, rowsa1))
    B = ((sidxb0, didxb0, rowsb0), (sidxb1, didxb1, rowsb1))
    c = lax.axis_index("c")
    s = lax.axis_index("s")
    base = c * HALF

    @pl.when(s < NS - 1)
    def _():
        pltpu.sync_copy(zrows.at[pl.ds(0, SSL)], acc.at[pl.ds(s * SSL, SSL)])

    @pl.when(s == NS - 1)
    def _():
        pltpu.sync_copy(zrows, acc.at[pl.ds((NS - 1) * SSL, SSL_LAST + 8)])

    pltpu.sync_copy(mode, modev)
    m = modev[...][0]

    @pl.when(m == 0)
    def _():
        pltpu.sync_copy(y.at[pl.ds(0, CHUNK)], rowsa0)

    plsc.subcore_barrier()

    nk = KBASE + jnp.where(s < KREM, 1, 0)

    def load_issue(k, bufs, sem):
        rb = (k * NS + s) * RPI
        for j in range(RPI):
            pltpu.sync_copy(dst2d.at[rb + j, pl.ds(0, CHUNK)], bufs[j][1])

        @pl.when(m > 0)
        def _():
            for j in range(RPI):
                pltpu.sync_copy(src2d.at[rb + j, pl.ds(0, CHUNK)], bufs[j][0])
            for j in range(RPI):
                pltpu.async_copy(y.at[bufs[j][0]], bufs[j][2], sem)

    def remap(bufs):
        trash = HALF + (lax.iota(jnp.int32, L) & 7)
        for j in range(RPI):
            didx = bufs[j][1]
            for g in range(CHUNK // L):
                dv = didx[pl.ds(g * L, L)] - base
                bad = (dv < 0) | (dv >= HALF)
                didx[pl.ds(g * L, L)] = jnp.where(bad, trash, dv)

    def wait_scatter(bufs, sem, scsem):
        @pl.when(m > 0)
        def _():
            for j in range(RPI):
                pltpu.make_async_copy(y.at[bufs[j][0]], bufs[j][2], sem).wait()
            for j in range(RPI):
                pltpu.async_copy(bufs[j][2], acc.at[bufs[j][1]], scsem,
                                 add=True)

        @pl.when(m == 0)
        def _():
            for j in range(RPI):
                pltpu.async_copy(rowsa0, acc.at[bufs[j][1]], scsem, add=True)

    def drain_scatter(bufs, scsem):
        @pl.when(m > 0)
        def _():
            for j in range(RPI):
                pltpu.make_async_copy(bufs[j][2], acc.at[bufs[j][1]],
                                      scsem).wait()

        @pl.when(m == 0)
        def _():
            for j in range(RPI):
                pltpu.make_async_copy(rowsa0, acc.at[bufs[j][1]],
                                      scsem).wait()

    def body(k2, carry):
        @pl.when(k2 > 0)
        def _():
            drain_scatter(A, scsema)
        load_issue(2 * k2, A, sema)

        @pl.when(k2 > 0)
        def _():
            drain_scatter(B, scsemb)
        load_issue(2 * k2 + 1, B, semb)
        remap(A)
        wait_scatter(A, sema, scsema)
        remap(B)
        wait_scatter(B, semb, scsemb)
        return carry

    nk2 = nk // 2
    lax.fori_loop(0, nk2, body, 0)

    @pl.when(nk2 > 0)
    def _():
        drain_scatter(A, scsema)
        drain_scatter(B, scsemb)

    @pl.when(nk % 2 == 1)
    def _():
        load_issue(nk - 1, A, sema)
        remap(A)

        @pl.when(m > 0)
        def _():
            for j in range(RPI):
                pltpu.make_async_copy(y.at[A[j][0]], A[j][2], sema).wait()
            for j in range(RPI):
                pltpu.sync_copy(A[j][2], acc.at[A[j][1]], add=True)

        @pl.when(m == 0)
        def _():
            for j in range(RPI):
                pltpu.sync_copy(rowsa0, acc.at[A[j][1]], add=True)

    plsc.subcore_barrier()

    @pl.when(s < NS - 1)
    def _():
        pltpu.sync_copy(acc.at[pl.ds(s * SSL, SSL)],
                        out.at[c, pl.ds(s * SSL, SSL)])

    @pl.when(s == NS - 1)
    def _():
        pltpu.sync_copy(acc.at[pl.ds((NS - 1) * SSL, SSL_LAST)],
                        out.at[c, pl.ds((NS - 1) * SSL, SSL_LAST)])


_sc_agg = pl.kernel(
    _sc_agg_body,
    mesh=plsc.VectorSubcoreMesh(core_axis_name="c", subcore_axis_name="s"),
    out_type=jax.ShapeDtypeStruct((NC, HALF, D), jnp.float32),
    scratch_types=(
        ([pltpu.VMEM((CHUNK,), jnp.int32)] * 4
         + [pltpu.VMEM((CHUNK, D), jnp.float32)] * 2) * 2
        + [pltpu.VMEM((L,), jnp.int32),
           pltpu.VMEM_SHARED((HPAD, D), jnp.float32),
           pltpu.SemaphoreType.DMA, pltpu.SemaphoreType.DMA,
           pltpu.SemaphoreType.DMA, pltpu.SemaphoreType.DMA]
    ),
)


def _lin1_body(x_ref, wl_ref, wr_ref, b_ref, y_ref, xr_ref):
    xb = x_ref[...]
    y_ref[...] = jnp.dot(xb, wl_ref[...], preferred_element_type=jnp.float32)
    xr_ref[...] = (
        jnp.dot(xb, wr_ref[...], preferred_element_type=jnp.float32)
        + b_ref[...]
    )


def _stage_body(p_ref, y1_ref, xr_ref, cnt_ref, wl_ref, wr_ref, b_ref,
                fc_ref, fr_ref, ytab2_ref, xr2_ref, cnt2_ref, t_ref):
    pb = p_ref[0]
    fc = fc_ref[...]
    fr = fr_ref[...]
    cnt2 = fc * (pb[:, :1] * jnp.ones((1, D), jnp.float32)) \
        + (1.0 - fc) * cnt_ref[...]
    cnt2_ref[...] = cnt2
    t = pb / jnp.maximum(cnt2[:, :1], 1.0) + xr_ref[...]
    t_ref[...] = t
    h = jnp.maximum(t, (1.0 - fr) * t)
    ynew = jnp.dot(h, wl_ref[...], preferred_element_type=jnp.float32)
    xrnew = (
        jnp.dot(h, wr_ref[...], preferred_element_type=jnp.float32)
        + b_ref[...]
    )
    ytab2_ref[...] = fc * y1_ref[...] + (1.0 - fc) * ynew
    xr2_ref[...] = fc * xr_ref[...] + (1.0 - fc) * xrnew


def _row_spec(shape):
    return pl.BlockSpec(shape, lambda i: (i, 0))


def _full_spec(shape):
    return pl.BlockSpec(shape, lambda i: (0, 0))


_tc_lin1 = pl.pallas_call(
    _lin1_body,
    grid=(N // BN,),
    in_specs=[_row_spec((BN, D)), _full_spec((D, D)), _full_spec((D, D)),
              _full_spec((1, D))],
    out_specs=[_row_spec((BN, D)), _row_spec((BN, D))],
    out_shape=[jax.ShapeDtypeStruct((N, D), jnp.float32),
               jax.ShapeDtypeStruct((N, D), jnp.float32)],
)

_tc_stage = pl.pallas_call(
    _stage_body,
    grid=(N // BN,),
    in_specs=[pl.BlockSpec((1, BN, D), lambda i: (i // BPH, i % BPH, 0)),
              _row_spec((BN, D)), _row_spec((BN, D)), _row_spec((BN, D)),
              _full_spec((D, D)), _full_spec((D, D)), _full_spec((1, D)),
              _full_spec((1, D)), _full_spec((1, D))],
    out_specs=[_row_spec((BN, D)), _row_spec((BN, D)), _row_spec((BN, D)),
               _row_spec((BN, D))],
    out_shape=[jax.ShapeDtypeStruct((N, D), jnp.float32),
               jax.ShapeDtypeStruct((N, D), jnp.float32),
               jax.ShapeDtypeStruct((N, D), jnp.float32),
               jax.ShapeDtypeStruct((N, D), jnp.float32)],
)


@jax.jit
def kernel(x, edge_index, W1l, W1r, b1, W2l, W2r, b2):
    src2d = edge_index[0].reshape(EC, CHUNK)
    dst2d = edge_index[1].reshape(EC, CHUNK)
    zrows = jnp.zeros((SSL_LAST + 8, D), jnp.float32)
    b1r = b1.reshape(1, D)
    b2r = b2.reshape(1, D)

    y1, xr1 = _tc_lin1(x, W1l, W1r, b1r)

    def step(i, carry):
        ytab, xr, cnt, _ = carry
        mode = jnp.where(i == 0, 0, 1) * jnp.ones((L,), jnp.int32)
        p = _sc_agg(src2d, dst2d, ytab, zrows, mode)
        fc = jnp.where(i == 0, 1.0, 0.0) * jnp.ones((1, D), jnp.float32)
        fr = jnp.where(i == 1, 1.0, 0.0) * jnp.ones((1, D), jnp.float32)
        ytab2, xr2, cnt2, t = _tc_stage(p, y1, xr, cnt, W2l, W2r, b2r,
                                        fc, fr)
        return (ytab2, xr2, cnt2, t)

    dummy = jnp.zeros((N, D), jnp.float32)
    onestab = jnp.ones((N, D), jnp.float32)
    _, _, _, out = lax.fori_loop(0, 3, step, (onestab, xr1, dummy, dummy))
    return out
